# baseline (speedup 1.0000x reference)
import functools
import jax
import jax.numpy as jnp
from jax import lax
from jax.experimental import pallas as pl
from jax.experimental.pallas import tpu as pltpu
from jax.experimental.pallas import tpu_sc as plsc

NC = 2
NS = 16
NW = NC * NS
B = 128
CPT = 80
EPT = B * CPT
EPAD = NW * EPT
TROWS = 32
H = B // 2


def _tbl_body(e1_ref, e2_ref, out_ref):
  c = lax.broadcasted_iota(jnp.int32, (TROWS, 1), 0)
  i6 = lax.broadcasted_iota(jnp.int32, (1, 6), 1)
  i3 = lax.broadcasted_iota(jnp.int32, (1, 3), 1)
  s1 = (c // 3 == i6).astype(jnp.float32)
  s2 = ((c % 3 == i3) & (c < 18)).astype(jnp.float32)
  out_ref[...] = (jnp.dot(s1, e1_ref[...], preferred_element_type=jnp.float32)
                  + jnp.dot(s2, e2_ref[...], preferred_element_type=jnp.float32))


def _tc_table(ee1, ee2):
  d = ee1.shape[1]
  return pl.pallas_call(
      _tbl_body,
      out_shape=jax.ShapeDtypeStruct((TROWS, d), jnp.float32),
  )(ee1, ee2)


def _sc_scatter(x, srcp, dstp, combop, tbl, n_pad):
  D = x.shape[1]
  rpt = n_pad // NS
  mesh = plsc.VectorSubcoreMesh(core_axis_name="c", subcore_axis_name="s")

  @functools.partial(
      pl.kernel,
      out_type=[
          jax.ShapeDtypeStruct((NC, n_pad, D), jnp.float32),
      ],
      mesh=mesh,
      scratch_types=[
          pltpu.VMEM((H,), jnp.int32),
          pltpu.VMEM((H,), jnp.int32),
          pltpu.VMEM((H,), jnp.int32),
          pltpu.VMEM((H,), jnp.int32),
          pltpu.VMEM((H,), jnp.int32),
          pltpu.VMEM((H,), jnp.int32),
          pltpu.VMEM((H,), jnp.int32),
          pltpu.VMEM((H,), jnp.int32),
          pltpu.VMEM((H,), jnp.int32),
          pltpu.VMEM((H,), jnp.int32),
          pltpu.VMEM((H,), jnp.int32),
          pltpu.VMEM((H,), jnp.int32),
          pltpu.VMEM((H, 128), jnp.float32),
          pltpu.VMEM((H, 128), jnp.float32),
          pltpu.VMEM((H, 128), jnp.float32),
          pltpu.VMEM((H, 128), jnp.float32),
          pltpu.VMEM_SHARED((n_pad, 128), jnp.float32),
          pltpu.SemaphoreType.DMA,
          pltpu.SemaphoreType.DMA,
          pltpu.SemaphoreType.DMA,
          pltpu.SemaphoreType.DMA,
          pltpu.SemaphoreType.DMA,
          pltpu.SemaphoreType.DMA,
          pltpu.SemaphoreType.DMA,
          pltpu.SemaphoreType.DMA,
      ],
  )
  def k(x_hbm, src_hbm, dst_hbm, combo_hbm, tbl_hbm, aggr_out,
        src_a0, src_a1, src_b0, src_b1, dst_a0, dst_a1, dst_b0, dst_b1,
        combo_a0, combo_a1, combo_b0, combo_b1,
        rx0_v, rx1_v, rt0_v, rt1_v, aggr_sh,
        gsem_x0, gsem_x1, gsem_t0, gsem_t1,
        ssem_x0, ssem_x1, ssem_t0, ssem_t1):
    c = lax.axis_index("c")
    s = lax.axis_index("s")
    wid = c * NS + s

    def zero_rows(i, carry):
      def zcol(j, carry2):
        rt0_v[i, pl.ds(j * 16, 16)] = jnp.zeros((16,), jnp.float32)
        return carry2
      return lax.fori_loop(0, 128 // 16, zcol, carry)
    lax.fori_loop(0, H, zero_rows, 0)

    def zero_shared(j, carry):
      pltpu.sync_copy(rt0_v, aggr_sh.at[pl.ds(s * rpt + j * H, H)])
      return carry
    lax.fori_loop(0, rpt // H, zero_shared, 0)

    plsc.subcore_barrier()

    def stage_idx(g, idx):
      base = 2 * (wid * CPT + g)
      pltpu.sync_copy(src_hbm.at[base], idx[0])
      pltpu.sync_copy(src_hbm.at[base + 1], idx[1])
      pltpu.sync_copy(dst_hbm.at[base], idx[2])
      pltpu.sync_copy(dst_hbm.at[base + 1], idx[3])
      pltpu.sync_copy(combo_hbm.at[base], idx[4])
      pltpu.sync_copy(combo_hbm.at[base + 1], idx[5])

    def gathers(idx):
      pltpu.async_copy(x_hbm.at[idx[0]], rx0_v, gsem_x0)
      pltpu.async_copy(x_hbm.at[idx[1]], rx1_v, gsem_x1)
      pltpu.async_copy(tbl_hbm.at[idx[4]], rt0_v, gsem_t0)
      pltpu.async_copy(tbl_hbm.at[idx[5]], rt1_v, gsem_t1)

    set_a = (src_a0, src_a1, dst_a0, dst_a1, combo_a0, combo_a1)
    set_b = (src_b0, src_b1, dst_b0, dst_b1, combo_b0, combo_b1)

    stage_idx(0, set_a)
    gathers(set_a)

    def pair(p, carry):
      for q, (idx, idx2) in enumerate([(set_a, set_b), (set_b, set_a)]):
        g = 2 * p + q
        pltpu.make_async_copy(x_hbm.at[idx[0]], rx0_v, gsem_x0).wait()
        pltpu.make_async_copy(x_hbm.at[idx[1]], rx1_v, gsem_x1).wait()
        pltpu.make_async_copy(tbl_hbm.at[idx[4]], rt0_v, gsem_t0).wait()
        pltpu.make_async_copy(tbl_hbm.at[idx[5]], rt1_v, gsem_t1).wait()
        pltpu.async_copy(rx0_v, aggr_sh.at[idx[2]], ssem_x0, add=True)
        pltpu.async_copy(rx1_v, aggr_sh.at[idx[3]], ssem_x1, add=True)
        pltpu.async_copy(rt0_v, aggr_sh.at[idx[2]], ssem_t0, add=True)
        pltpu.async_copy(rt1_v, aggr_sh.at[idx[3]], ssem_t1, add=True)
        @pl.when(g + 1 < CPT)
        def _():
          stage_idx(g + 1, idx2)
        pltpu.make_async_copy(rx0_v, aggr_sh.at[idx[2]], ssem_x0).wait()
        pltpu.make_async_copy(rx1_v, aggr_sh.at[idx[3]], ssem_x1).wait()
        pltpu.make_async_copy(rt0_v, aggr_sh.at[idx[2]], ssem_t0).wait()
        pltpu.make_async_copy(rt1_v, aggr_sh.at[idx[3]], ssem_t1).wait()
        @pl.when(g + 1 < CPT)
        def _():
          gathers(idx2)
      return carry
    lax.fori_loop(0, CPT // 2, pair, 0)

    plsc.subcore_barrier()

    pltpu.sync_copy(aggr_sh.at[pl.ds(s * rpt, rpt)],
                    aggr_out.at[c, pl.ds(s * rpt, rpt)])

  return k(x, srcp, dstp, combop, tbl)


def _mlp_body(p0, p1, x_ref, tbl, w1, b1, w2, b2, out_ref):
  self_row = tbl[12:13, :]
  aggr = p0[...] + p1[...] + x_ref[...] + self_row
  h = jnp.dot(aggr, w1[...], preferred_element_type=jnp.float32) + b1[...]
  h = jnp.maximum(h, 0.0)
  out_ref[...] = jnp.dot(h, w2[...], preferred_element_type=jnp.float32) + b2[...]


def _tc_mlp(p0, p1, x, tbl, w1, b1, w2, b2):
  n, d = x.shape
  r = 1000
  blk = lambda rr, cc: pl.BlockSpec((rr, cc), lambda i: (i, 0))
  full = lambda rr, cc: pl.BlockSpec((rr, cc), lambda i: (0, 0))
  return pl.pallas_call(
      _mlp_body,
      grid=(n // r,),
      in_specs=[
          blk(r, d), blk(r, d), blk(r, d),
          full(TROWS, d), full(d, w1.shape[1]), full(1, b1.shape[1]),
          full(w2.shape[0], d), full(1, d),
      ],
      out_specs=blk(r, d),
      out_shape=jax.ShapeDtypeStruct((n, d), jnp.float32),
  )(p0, p1, x, tbl, w1, b1, w2, b2)


@jax.jit
def kernel(x, edge_index, edge_attr, ee1, ee2, W1, b1, W2, b2):
  n, d = x.shape
  e = edge_index.shape[1]
  n_pad = 10240
  pad = EPAD - e

  src = jnp.pad(edge_index[0], (0, pad)).reshape(NW * CPT * 2, H)
  dst = jnp.pad(edge_index[1], (0, pad), constant_values=n).reshape(NW * CPT * 2, H)
  ar = jnp.arange(EPAD, dtype=jnp.int32)
  eidx = (ar + (ar // EPT) * 16) % 512
  combo = jnp.pad(edge_attr[:, 0] * 3 + edge_attr[:, 1], (0, pad))
  combo = (combo * 512 + eidx).reshape(NW * CPT * 2, H)

  tbl = _tc_table(ee1, ee2)
  tbl_rep = jnp.repeat(tbl, 512, axis=0)
  (aggr_p,) = _sc_scatter(x, src, dst, combo, tbl_rep, n_pad)

  return _tc_mlp(aggr_p[0, :n], aggr_p[1, :n], x, tbl,
                 W1, b1.reshape(1, -1), W2, b2.reshape(1, -1))
